# initial kernel scaffold (unmeasured)
import jax
import jax.numpy as jnp
from jax import lax
from jax.experimental import pallas as pl
from jax.experimental.pallas import tpu as pltpu


def kernel(
    x,
):
    def body(*refs):
        pass

    out_shape = jax.ShapeDtypeStruct(..., jnp.float32)
    return pl.pallas_call(body, out_shape=out_shape)(...)


# baseline (device time: 9834 ns/iter reference)
import jax
import jax.numpy as jnp
from jax import lax
from jax.experimental import pallas as pl
from jax.experimental.pallas import tpu as pltpu

N_DEV = 16


def kernel(x):
    m, n = x.shape

    def body(x_ref, out_ref, gather_ref, send_sems, recv_sems):
        my = lax.axis_index("i")

        barrier = pltpu.get_barrier_semaphore()
        for h in range(1, N_DEV):
            peer = lax.rem(my + h, N_DEV)
            pl.semaphore_signal(
                barrier, inc=1,
                device_id=(peer,), device_id_type=pl.DeviceIdType.MESH,
            )
        pl.semaphore_wait(barrier, N_DEV - 1)

        gather_ref[pl.ds(my, 1), :] = jnp.sum(x_ref[:, :], axis=0, keepdims=True)

        rdmas = []
        for h in range(1, N_DEV):
            peer = lax.rem(my + h, N_DEV)
            rdma = pltpu.make_async_remote_copy(
                src_ref=gather_ref.at[pl.ds(my, 1)],
                dst_ref=gather_ref.at[pl.ds(my, 1)],
                send_sem=send_sems.at[h - 1],
                recv_sem=recv_sems.at[h - 1],
                device_id=(peer,),
                device_id_type=pl.DeviceIdType.MESH,
            )
            rdma.start()
            rdmas.append(rdma)

        for rdma in rdmas:
            rdma.wait_recv()
        for rdma in rdmas:
            rdma.wait_send()

        out_ref[...] = jnp.sum(gather_ref[...], axis=0, keepdims=True)

    return pl.pallas_call(
        body,
        out_shape=jax.ShapeDtypeStruct((1, n), x.dtype),
        in_specs=[pl.BlockSpec(memory_space=pltpu.VMEM)],
        out_specs=pl.BlockSpec(memory_space=pltpu.VMEM),
        scratch_shapes=[
            pltpu.VMEM((N_DEV, n), x.dtype),
            pltpu.SemaphoreType.DMA((N_DEV - 1,)),
            pltpu.SemaphoreType.DMA((N_DEV - 1,)),
        ],
        compiler_params=pltpu.CompilerParams(collective_id=0),
    )(x)


# device time: 9326 ns/iter; 1.0545x vs baseline; 1.0545x over previous
import jax
import jax.numpy as jnp
from jax import lax
from jax.experimental import pallas as pl
from jax.experimental.pallas import tpu as pltpu

N_DEV = 16


def kernel(x):
    m, n = x.shape

    def body(x_ref, out_ref, gather_ref, send_sems, recv_sems):
        my = lax.axis_index("i")

        barrier = pltpu.get_barrier_semaphore()
        for h in range(1, N_DEV):
            peer = lax.rem(my + h, N_DEV)
            pl.semaphore_signal(
                barrier, inc=1,
                device_id=(peer,), device_id_type=pl.DeviceIdType.MESH,
            )

        gather_ref[pl.ds(my, 1), :] = jnp.sum(x_ref[:, :], axis=0, keepdims=True)

        pl.semaphore_wait(barrier, N_DEV - 1)

        rdmas = []
        for h in range(1, N_DEV):
            peer = lax.rem(my + h, N_DEV)
            rdma = pltpu.make_async_remote_copy(
                src_ref=gather_ref.at[pl.ds(my, 1)],
                dst_ref=gather_ref.at[pl.ds(my, 1)],
                send_sem=send_sems.at[h - 1],
                recv_sem=recv_sems.at[h - 1],
                device_id=(peer,),
                device_id_type=pl.DeviceIdType.MESH,
            )
            rdma.start()
            rdmas.append(rdma)

        for rdma in rdmas:
            rdma.wait_recv()

        out_ref[...] = jnp.sum(gather_ref[...], axis=0, keepdims=True)

        for rdma in rdmas:
            rdma.wait_send()

    return pl.pallas_call(
        body,
        out_shape=jax.ShapeDtypeStruct((1, n), x.dtype),
        in_specs=[pl.BlockSpec(memory_space=pltpu.VMEM)],
        out_specs=pl.BlockSpec(memory_space=pltpu.VMEM),
        scratch_shapes=[
            pltpu.VMEM((N_DEV, n), x.dtype),
            pltpu.SemaphoreType.DMA((N_DEV - 1,)),
            pltpu.SemaphoreType.DMA((N_DEV - 1,)),
        ],
        compiler_params=pltpu.CompilerParams(collective_id=0),
    )(x)
